# Initial kernel scaffold; baseline (speedup 1.0000x reference)
import jax
import jax.numpy as jnp
from jax.experimental import pallas as pl


def kernel(x, embed):
    raise NotImplementedError("write your pallas kernel here")


# speedup vs baseline: 1.2212x; 1.2212x over previous
import functools

import jax
import jax.numpy as jnp
from jax import lax
from jax.experimental import pallas as pl
from jax.experimental.pallas import tpu as pltpu
from jax.experimental.pallas import tpu_sc as plsc

DIM = 256
CODES = 8192
TOKENS = 16384

M_BLOCK = 512
N_BLOCK = 1024
M_TILES = TOKENS // M_BLOCK
N_TILES = CODES // N_BLOCK

SC_CORES = 2
SC_SUBCORES = 16
SC_WORKERS = SC_CORES * SC_SUBCORES
ROWS_PER_WORKER = TOKENS // SC_WORKERS
GATHER_CHUNK = 256

_BIG = 2**30


_CHUNKS = ((0, 2736), (2736, 2736), (5472, 2720))


def _argmin_dist_body(x_ref, e_ref, x2_ref, e2_ref, out_ref):
    x = x_ref[...]
    x2 = x2_ref[0]

    bv = None
    bi = None
    for off, size in _CHUNKS:
        e = e_ref[pl.ds(off, size), :]
        e2 = e2_ref[pl.ds(off, size), :]
        xe_t = lax.dot_general(e, x, (((1,), (1,)), ((), ())),
                               preferred_element_type=jnp.float32)
        dist = -((x2 - 2.0 * xe_t) + e2)

        m = jnp.max(dist, axis=0, keepdims=True)
        row_ids = (lax.broadcasted_iota(jnp.int32, (size, M_BLOCK), 0) + off)
        cand = jnp.where(dist == m, row_ids, _BIG)
        idx = jnp.min(cand, axis=0, keepdims=True)

        if bv is None:
            bv, bi = m, idx
        else:
            better = m > bv
            bi = jnp.where(better, idx, bi)
            bv = jnp.maximum(bv, m)
        bv = bv.astype(jnp.bfloat16).astype(jnp.float32)

    out_ref[0] = bi


def _nearest_code_indices(flat_x, embed, x2, e2):
    x2_3d = x2.reshape(M_TILES, 1, M_BLOCK)
    e2_2d = e2.reshape(CODES, 1)
    out = pl.pallas_call(
        _argmin_dist_body,
        grid=(M_TILES,),
        in_specs=[
            pl.BlockSpec((M_BLOCK, DIM), lambda i: (i, 0)),
            pl.BlockSpec((CODES, DIM), lambda i: (0, 0)),
            pl.BlockSpec((1, 1, M_BLOCK), lambda i: (i, 0, 0)),
            pl.BlockSpec((CODES, 1), lambda i: (0, 0)),
        ],
        out_specs=pl.BlockSpec((1, 1, M_BLOCK), lambda i: (i, 0, 0)),
        out_shape=jax.ShapeDtypeStruct((M_TILES, 1, M_BLOCK), jnp.int32),
    )(flat_x, embed, x2_3d, e2_2d)
    return out.reshape(TOKENS)


@functools.cache
def _make_sc_gather():
    mesh = plsc.VectorSubcoreMesh(core_axis_name="c", subcore_axis_name="s")

    @functools.partial(
        pl.kernel,
        mesh=mesh,
        out_type=jax.ShapeDtypeStruct((TOKENS, DIM), jnp.float32),
        scratch_types=[
            pltpu.VMEM((GATHER_CHUNK,), jnp.int32),
            pltpu.VMEM((GATHER_CHUNK, DIM), jnp.float32),
            pltpu.SemaphoreType.DMA,
        ],
    )
    def gather_rows(table_hbm, idx_hbm, out_hbm, idx_v, rows_v, sem):
        wid = lax.axis_index("s") * SC_CORES + lax.axis_index("c")
        for c in range(ROWS_PER_WORKER // GATHER_CHUNK):
            base = wid * ROWS_PER_WORKER + c * GATHER_CHUNK
            pltpu.sync_copy(idx_hbm.at[pl.ds(base, GATHER_CHUNK)], idx_v)
            pltpu.async_copy(table_hbm.at[idx_v], rows_v, sem).wait()
            pltpu.sync_copy(rows_v, out_hbm.at[pl.ds(base, GATHER_CHUNK)])

    return gather_rows


def kernel(x, embed):
    shape = x.shape
    flat_x = x.reshape(-1, shape[-1])
    x2 = jnp.sum(flat_x ** 2, axis=1)
    e2 = jnp.sum(embed ** 2, axis=1)
    idx = _nearest_code_indices(flat_x, embed, x2, e2)
    quantize = _make_sc_gather()(embed, idx).reshape(shape)
    num_replace = jnp.array(0, dtype=jnp.int32)
    return (quantize, num_replace)


# speedup vs baseline: 1.4078x; 1.1528x over previous
import functools

import jax
import jax.numpy as jnp
from jax import lax
from jax.experimental import pallas as pl
from jax.experimental.pallas import tpu as pltpu
from jax.experimental.pallas import tpu_sc as plsc

DIM = 256
CODES = 8192
TOKENS = 16384

M_BLOCK = 512
N_BLOCK = 1024
M_TILES = TOKENS // M_BLOCK
N_TILES = CODES // N_BLOCK

SC_CORES = 2
SC_SUBCORES = 16
SC_WORKERS = SC_CORES * SC_SUBCORES
ROWS_PER_WORKER = TOKENS // SC_WORKERS
GATHER_CHUNK = 256

_BIG = 2**30


_CHUNKS = ((0, 2736), (2736, 2736), (5472, 2720))


def _argmin_dist_body(x_ref, e_ref, x2_ref, e2_ref, out_ref):
    xd = x_ref[...] + x_ref[...]
    x2 = x2_ref[0]

    bv = None
    bi = None
    for off, size in _CHUNKS:
        e = e_ref[pl.ds(off, size), :]
        e2 = e2_ref[pl.ds(off, size), :]
        xe2 = lax.dot_general(e, xd, (((1,), (1,)), ((), ())),
                              preferred_element_type=jnp.float32)
        dist = (xe2 - x2) - e2

        m = jnp.max(dist, axis=0, keepdims=True)
        row_ids = (lax.broadcasted_iota(jnp.int32, (size, M_BLOCK), 0) + off)
        cand = jnp.where(dist == m, row_ids, _BIG)
        idx = jnp.min(cand, axis=0, keepdims=True)

        if bv is None:
            bv, bi = m, idx
        else:
            better = m > bv
            bi = jnp.where(better, idx, bi)
            bv = jnp.maximum(bv, m)
        bv = bv.astype(jnp.bfloat16).astype(jnp.float32)

    out_ref[0] = bi


def _nearest_code_indices(flat_x, embed, x2, e2):
    x2_3d = x2.reshape(M_TILES, 1, M_BLOCK)
    e2_2d = e2.reshape(CODES, 1)
    out = pl.pallas_call(
        _argmin_dist_body,
        grid=(M_TILES,),
        in_specs=[
            pl.BlockSpec((M_BLOCK, DIM), lambda i: (i, 0)),
            pl.BlockSpec((CODES, DIM), lambda i: (0, 0)),
            pl.BlockSpec((1, 1, M_BLOCK), lambda i: (i, 0, 0)),
            pl.BlockSpec((CODES, 1), lambda i: (0, 0)),
        ],
        out_specs=pl.BlockSpec((1, 1, M_BLOCK), lambda i: (i, 0, 0)),
        out_shape=jax.ShapeDtypeStruct((M_TILES, 1, M_BLOCK), jnp.int32),
    )(flat_x, embed, x2_3d, e2_2d)
    return out.reshape(TOKENS)


@functools.cache
def _make_sc_gather():
    mesh = plsc.VectorSubcoreMesh(core_axis_name="c", subcore_axis_name="s")

    @functools.partial(
        pl.kernel,
        mesh=mesh,
        out_type=jax.ShapeDtypeStruct((TOKENS, DIM), jnp.float32),
        scratch_types=[
            pltpu.VMEM((GATHER_CHUNK,), jnp.int32),
            pltpu.VMEM((GATHER_CHUNK, DIM), jnp.float32),
            pltpu.SemaphoreType.DMA,
        ],
    )
    def gather_rows(table_hbm, idx_hbm, out_hbm, idx_v, rows_v, sem):
        wid = lax.axis_index("s") * SC_CORES + lax.axis_index("c")
        for c in range(ROWS_PER_WORKER // GATHER_CHUNK):
            base = wid * ROWS_PER_WORKER + c * GATHER_CHUNK
            pltpu.sync_copy(idx_hbm.at[pl.ds(base, GATHER_CHUNK)], idx_v)
            pltpu.async_copy(table_hbm.at[idx_v], rows_v, sem).wait()
            pltpu.sync_copy(rows_v, out_hbm.at[pl.ds(base, GATHER_CHUNK)])

    return gather_rows


def kernel(x, embed):
    shape = x.shape
    flat_x = x.reshape(-1, shape[-1])
    x2 = jnp.sum(flat_x ** 2, axis=1)
    e2 = jnp.sum(embed ** 2, axis=1)
    idx = _nearest_code_indices(flat_x, embed, x2, e2)
    quantize = _make_sc_gather()(embed, idx).reshape(shape)
    num_replace = jnp.array(0, dtype=jnp.int32)
    return (quantize, num_replace)


# speedup vs baseline: 1.7584x; 1.2490x over previous
import functools

import jax
import jax.numpy as jnp
from jax import lax
from jax.experimental import pallas as pl
from jax.experimental.pallas import tpu as pltpu
from jax.experimental.pallas import tpu_sc as plsc

DIM = 256
CODES = 8192
TOKENS = 16384

M_BLOCK = 512
N_BLOCK = 1024
M_TILES = TOKENS // M_BLOCK
N_TILES = CODES // N_BLOCK

SC_CORES = 2
SC_SUBCORES = 16
SC_WORKERS = SC_CORES * SC_SUBCORES
ROWS_PER_WORKER = TOKENS // SC_WORKERS
GATHER_CHUNK = 256

_BIG = 2**30


_CHUNKS = ((0, 2736), (2736, 2736), (5472, 2720))


def _argmin_dist_body(x_ref, e_ref, x2_ref, e2_ref, out_ref):
    xd = x_ref[...] + x_ref[...]
    x2 = x2_ref[0]

    bv = None
    bi = None
    for off, size in _CHUNKS:
        e = e_ref[pl.ds(off, size), :]
        e2 = e2_ref[pl.ds(off, size), :]
        xe2 = lax.dot_general(e, xd, (((1,), (1,)), ((), ())),
                              preferred_element_type=jnp.float32)
        dist = (xe2 - x2) - e2

        m = jnp.max(dist, axis=0, keepdims=True)
        idx = (jnp.argmax(dist, axis=0).astype(jnp.int32)[None, :] + off)

        if bv is None:
            bv, bi = m, idx
        else:
            better = m > bv
            bi = jnp.where(better, idx, bi)
            bv = jnp.maximum(bv, m)
        bv = bv.astype(jnp.bfloat16).astype(jnp.float32)

    out_ref[0] = bi


def _nearest_code_indices(flat_x, embed, x2, e2):
    x2_3d = x2.reshape(M_TILES, 1, M_BLOCK)
    e2_2d = e2.reshape(CODES, 1)
    out = pl.pallas_call(
        _argmin_dist_body,
        grid=(M_TILES,),
        in_specs=[
            pl.BlockSpec((M_BLOCK, DIM), lambda i: (i, 0)),
            pl.BlockSpec((CODES, DIM), lambda i: (0, 0)),
            pl.BlockSpec((1, 1, M_BLOCK), lambda i: (i, 0, 0)),
            pl.BlockSpec((CODES, 1), lambda i: (0, 0)),
        ],
        out_specs=pl.BlockSpec((1, 1, M_BLOCK), lambda i: (i, 0, 0)),
        out_shape=jax.ShapeDtypeStruct((M_TILES, 1, M_BLOCK), jnp.int32),
    )(flat_x, embed, x2_3d, e2_2d)
    return out.reshape(TOKENS)


@functools.cache
def _make_sc_gather():
    mesh = plsc.VectorSubcoreMesh(core_axis_name="c", subcore_axis_name="s")

    @functools.partial(
        pl.kernel,
        mesh=mesh,
        out_type=jax.ShapeDtypeStruct((TOKENS, DIM), jnp.float32),
        scratch_types=[
            pltpu.VMEM((GATHER_CHUNK,), jnp.int32),
            pltpu.VMEM((GATHER_CHUNK, DIM), jnp.float32),
            pltpu.SemaphoreType.DMA,
        ],
    )
    def gather_rows(table_hbm, idx_hbm, out_hbm, idx_v, rows_v, sem):
        wid = lax.axis_index("s") * SC_CORES + lax.axis_index("c")
        for c in range(ROWS_PER_WORKER // GATHER_CHUNK):
            base = wid * ROWS_PER_WORKER + c * GATHER_CHUNK
            pltpu.sync_copy(idx_hbm.at[pl.ds(base, GATHER_CHUNK)], idx_v)
            pltpu.async_copy(table_hbm.at[idx_v], rows_v, sem).wait()
            pltpu.sync_copy(rows_v, out_hbm.at[pl.ds(base, GATHER_CHUNK)])

    return gather_rows


def kernel(x, embed):
    shape = x.shape
    flat_x = x.reshape(-1, shape[-1])
    x2 = jnp.sum(flat_x ** 2, axis=1)
    e2 = jnp.sum(embed ** 2, axis=1)
    idx = _nearest_code_indices(flat_x, embed, x2, e2)
    quantize = _make_sc_gather()(embed, idx).reshape(shape)
    num_replace = jnp.array(0, dtype=jnp.int32)
    return (quantize, num_replace)


# speedup vs baseline: 1.7948x; 1.0207x over previous
import functools

import jax
import jax.numpy as jnp
from jax import lax
from jax.experimental import pallas as pl
from jax.experimental.pallas import tpu as pltpu
from jax.experimental.pallas import tpu_sc as plsc

DIM = 256
CODES = 8192
TOKENS = 16384

M_BLOCK = 1024
N_BLOCK = 1024
M_TILES = TOKENS // M_BLOCK
N_TILES = CODES // N_BLOCK

SC_CORES = 2
SC_SUBCORES = 16
SC_WORKERS = SC_CORES * SC_SUBCORES
ROWS_PER_WORKER = TOKENS // SC_WORKERS
GATHER_CHUNK = 256

_BIG = 2**30


_CHUNKS = ((0, 2736), (2736, 2736), (5472, 2720))


def _argmin_dist_body(x_ref, e_ref, x2_ref, e2_ref, out_ref):
    xd = x_ref[...] + x_ref[...]
    x2 = x2_ref[0]

    bv = None
    bi = None
    for off, size in _CHUNKS:
        e = e_ref[pl.ds(off, size), :]
        e2 = e2_ref[pl.ds(off, size), :]
        xe2 = lax.dot_general(e, xd, (((1,), (1,)), ((), ())),
                              preferred_element_type=jnp.float32)
        dist = (xe2 - x2) - e2

        m = jnp.max(dist, axis=0, keepdims=True)
        idx = (jnp.argmax(dist, axis=0).astype(jnp.int32)[None, :] + off)

        if bv is None:
            bv, bi = m, idx
        else:
            better = m > bv
            bi = jnp.where(better, idx, bi)
            bv = jnp.maximum(bv, m)
        bv = bv.astype(jnp.bfloat16).astype(jnp.float32)

    out_ref[0] = bi


def _nearest_code_indices(flat_x, embed, x2, e2):
    x2_3d = x2.reshape(M_TILES, 1, M_BLOCK)
    e2_2d = e2.reshape(CODES, 1)
    out = pl.pallas_call(
        _argmin_dist_body,
        grid=(M_TILES,),
        in_specs=[
            pl.BlockSpec((M_BLOCK, DIM), lambda i: (i, 0)),
            pl.BlockSpec((CODES, DIM), lambda i: (0, 0)),
            pl.BlockSpec((1, 1, M_BLOCK), lambda i: (i, 0, 0)),
            pl.BlockSpec((CODES, 1), lambda i: (0, 0)),
        ],
        out_specs=pl.BlockSpec((1, 1, M_BLOCK), lambda i: (i, 0, 0)),
        out_shape=jax.ShapeDtypeStruct((M_TILES, 1, M_BLOCK), jnp.int32),
    )(flat_x, embed, x2_3d, e2_2d)
    return out.reshape(TOKENS)


@functools.cache
def _make_sc_gather():
    mesh = plsc.VectorSubcoreMesh(core_axis_name="c", subcore_axis_name="s")

    @functools.partial(
        pl.kernel,
        mesh=mesh,
        out_type=jax.ShapeDtypeStruct((TOKENS, DIM), jnp.float32),
        scratch_types=[
            pltpu.VMEM((GATHER_CHUNK,), jnp.int32),
            pltpu.VMEM((GATHER_CHUNK, DIM), jnp.float32),
            pltpu.SemaphoreType.DMA,
        ],
    )
    def gather_rows(table_hbm, idx_hbm, out_hbm, idx_v, rows_v, sem):
        wid = lax.axis_index("s") * SC_CORES + lax.axis_index("c")
        for c in range(ROWS_PER_WORKER // GATHER_CHUNK):
            base = wid * ROWS_PER_WORKER + c * GATHER_CHUNK
            pltpu.sync_copy(idx_hbm.at[pl.ds(base, GATHER_CHUNK)], idx_v)
            pltpu.async_copy(table_hbm.at[idx_v], rows_v, sem).wait()
            pltpu.sync_copy(rows_v, out_hbm.at[pl.ds(base, GATHER_CHUNK)])

    return gather_rows


def kernel(x, embed):
    shape = x.shape
    flat_x = x.reshape(-1, shape[-1])
    x2 = jnp.sum(flat_x ** 2, axis=1)
    e2 = jnp.sum(embed ** 2, axis=1)
    idx = _nearest_code_indices(flat_x, embed, x2, e2)
    quantize = _make_sc_gather()(embed, idx).reshape(shape)
    num_replace = jnp.array(0, dtype=jnp.int32)
    return (quantize, num_replace)
